# Initial kernel scaffold; baseline (speedup 1.0000x reference)
import jax
import jax.numpy as jnp
from jax.experimental import pallas as pl


def kernel(x_node, x_edge, edge_index, node2graph, W_node, b_node, W_edge, b_edge, W_msg, b_msg, W_upd, b_upd, Wi0, Wh0, b0, Wi1, Wh1, b1, Wm1, bm1, Wm2, bm2):
    raise NotImplementedError("write your pallas kernel here")


# speedup vs baseline: 4.4284x; 4.4284x over previous
import functools

import jax
import jax.numpy as jnp
from jax import lax
from jax.experimental import pallas as pl
from jax.experimental.pallas import tpu as pltpu
from jax.experimental.pallas import tpu_sc as plsc

NN = 10000
NE = 320000
DN = 128
DE = 16
H = 128
HE = 16
NL = 3
NG = 64
NI = 6

NC = 2
NS = 16
NW = NC * NS
EPW = NE // NW
CH = 40
NCH = EPW // CH
NPASS = 10
CPP = NCH // NPASS
RPS = 640
RPS_LAST = NN - RPS * (NS - 1)

_f32 = jnp.float32


def _sc_edge_body(p_hbm, ec_hbm, src_hbm, dst_hbm, zero_hbm, out_hbm,
                  src_v, dst_v, ec_v, g_v, acc_sh, ec_s0, ec_s1, g_s0, g_s1):
    c = lax.axis_index("c")
    s = lax.axis_index("s")
    wid = c * NS + s
    ebase = wid * EPW
    stripe = pl.multiple_of(s * RPS, 8)

    @pl.when(s < NS - 1)
    def _():
        pltpu.sync_copy(zero_hbm.at[pl.ds(stripe, RPS)],
                        acc_sh.at[pl.ds(stripe, RPS)])

    @pl.when(s == NS - 1)
    def _():
        pltpu.sync_copy(zero_hbm.at[pl.ds(RPS * (NS - 1), RPS_LAST)],
                        acc_sh.at[pl.ds(RPS * (NS - 1), RPS_LAST)])

    plsc.subcore_barrier()

    ec_sems = (ec_s0, ec_s1)
    g_sems = (g_s0, g_s1)

    def ppass(p, carry):
        pltpu.sync_copy(src_hbm.at[wid, p], src_v)
        pltpu.sync_copy(dst_hbm.at[wid, p], dst_v)
        pbase = ebase + p * (CPP * CH)

        def issue(j, b):
            off = pl.multiple_of(pbase + j * CH, 8)
            pltpu.async_copy(ec_hbm.at[pl.ds(off, CH)], ec_v.at[b],
                             ec_sems[b])
            pltpu.async_copy(p_hbm.at[src_v.at[j]], g_v.at[b], g_sems[b])

        def process(j, b):
            pltpu.make_async_copy(ec_hbm.at[pl.ds(ebase, CH)], ec_v.at[b],
                                  ec_sems[b]).wait()
            pltpu.make_async_copy(p_hbm.at[src_v.at[j]], g_v.at[b],
                                  g_sems[b]).wait()

            def row(r, cr):
                for q in range(H // 16):
                    sl = pl.ds(q * 16, 16)
                    ec_v[b, r, sl] = jnp.maximum(
                        ec_v[b, r, sl] + g_v[b, r, sl], 0.0)
                return cr

            lax.fori_loop(0, CH, row, 0)
            pltpu.sync_copy(ec_v.at[b], acc_sh.at[dst_v.at[j]], add=True)

            @pl.when(j + 2 < CPP)
            def _():
                issue(j + 2, b)

        issue(0, 0)
        issue(1, 1)

        def step(t, cr):
            process(2 * t, 0)
            process(2 * t + 1, 1)
            return cr

        lax.fori_loop(0, CPP // 2, step, 0)
        process(CPP - 1, (CPP - 1) % 2)
        return carry

    lax.fori_loop(0, NPASS, ppass, 0)
    plsc.subcore_barrier()

    @pl.when(s < NS - 1)
    def _():
        pltpu.sync_copy(acc_sh.at[pl.ds(stripe, RPS)],
                        out_hbm.at[c, pl.ds(stripe, RPS)])

    @pl.when(s == NS - 1)
    def _():
        pltpu.sync_copy(acc_sh.at[pl.ds(RPS * (NS - 1), RPS_LAST)],
                        out_hbm.at[c, pl.ds(RPS * (NS - 1), RPS_LAST)])


@functools.cache
def _sc_edge_kernel():
    return pl.kernel(
        _sc_edge_body,
        out_type=jax.ShapeDtypeStruct((NC, NN, H), _f32),
        mesh=plsc.VectorSubcoreMesh(core_axis_name="c", subcore_axis_name="s",
                                    num_cores=NC, num_subcores=NS),
        scratch_types=[
            pltpu.VMEM((CPP, CH), jnp.int32),
            pltpu.VMEM((CPP, CH), jnp.int32),
            pltpu.VMEM((2, CH, H), _f32),
            pltpu.VMEM((2, CH, H), _f32),
            pltpu.VMEM_SHARED((NN, H), _f32),
            pltpu.SemaphoreType.DMA,
            pltpu.SemaphoreType.DMA,
            pltpu.SemaphoreType.DMA,
            pltpu.SemaphoreType.DMA,
        ],
    )


def _sc_edge(p, ec_l, src_r, dst_r, zeros_nh):
    return _sc_edge_kernel()(p, ec_l, src_r, dst_r, zeros_nh)


_BE = 4000


def _edge_pre_body(xe, we, be, wm, bm, out):
    e = jnp.maximum(
        jnp.dot(xe[...], we[...], preferred_element_type=_f32) + be[...], 0.0)
    for l in range(NL):
        out[l] = jnp.dot(e, wm[l], preferred_element_type=_f32) + bm[l]


def _edge_pre(x_edge, W_edge, b_edge, Wm_e, bm):
    return pl.pallas_call(
        _edge_pre_body,
        grid=(NE // _BE,),
        in_specs=[
            pl.BlockSpec((_BE, DE), lambda i: (i, 0)),
            pl.BlockSpec((DE, HE), lambda i: (0, 0)),
            pl.BlockSpec((1, HE), lambda i: (0, 0)),
            pl.BlockSpec((NL, HE, H), lambda i: (0, 0, 0)),
            pl.BlockSpec((NL, 1, H), lambda i: (0, 0, 0)),
        ],
        out_specs=pl.BlockSpec((NL, _BE, H), lambda i: (0, i, 0)),
        out_shape=jax.ShapeDtypeStruct((NL, NE, H), _f32),
    )(x_edge, W_edge, b_edge, Wm_e, bm)


_BN = 2000


def _node0_body(xn, wn, bn, a, h_out, p_out):
    h = jnp.maximum(
        jnp.dot(xn[...], wn[...], preferred_element_type=_f32) + bn[...], 0.0)
    h_out[...] = h
    p_out[...] = jnp.dot(h, a[...], preferred_element_type=_f32)


def _node0(x_node, W_node, b_node, A0):
    return pl.pallas_call(
        _node0_body,
        grid=(NN // _BN,),
        in_specs=[
            pl.BlockSpec((_BN, DN), lambda i: (i, 0)),
            pl.BlockSpec((DN, H), lambda i: (0, 0)),
            pl.BlockSpec((1, H), lambda i: (0, 0)),
            pl.BlockSpec((H, H), lambda i: (0, 0)),
        ],
        out_specs=[
            pl.BlockSpec((_BN, H), lambda i: (i, 0)),
            pl.BlockSpec((_BN, H), lambda i: (i, 0)),
        ],
        out_shape=[
            jax.ShapeDtypeStruct((NN, H), _f32),
            jax.ShapeDtypeStruct((NN, H), _f32),
        ],
    )(x_node, W_node, b_node, A0)


def _upd_body(h, acc, wu, bu, a, hn_out, p_out):
    agg = acc[0] + acc[1]
    hn = jnp.maximum(
        h[...] + jnp.dot(agg, wu[...], preferred_element_type=_f32) + bu[...],
        0.0)
    hn_out[...] = hn
    p_out[...] = jnp.dot(hn, a[...], preferred_element_type=_f32)


def _upd(h, acc, Wu, bu, Anext):
    return pl.pallas_call(
        _upd_body,
        grid=(NN // _BN,),
        in_specs=[
            pl.BlockSpec((_BN, H), lambda i: (i, 0)),
            pl.BlockSpec((NC, _BN, H), lambda i: (0, i, 0)),
            pl.BlockSpec((H, H), lambda i: (0, 0)),
            pl.BlockSpec((1, H), lambda i: (0, 0)),
            pl.BlockSpec((H, H), lambda i: (0, 0)),
        ],
        out_specs=[
            pl.BlockSpec((_BN, H), lambda i: (i, 0)),
            pl.BlockSpec((_BN, H), lambda i: (i, 0)),
        ],
        out_shape=[
            jax.ShapeDtypeStruct((NN, H), _f32),
            jax.ShapeDtypeStruct((NN, H), _f32),
        ],
    )(h, acc, Wu, bu, Anext)


def _sigmoid(x):
    return 1.0 / (1.0 + jnp.exp(-x))


def _final_body(h_in, acc, wu, bu, n2g, wi0, wh0, b0, wi1, wh1, b1,
                wm1, bm1, wm2, bm2, out):
    agg = acc[0] + acc[1]
    h = jnp.maximum(
        h_in[...] + jnp.dot(agg, wu[...], preferred_element_type=_f32)
        + bu[...], 0.0)
    gid = lax.broadcasted_iota(jnp.int32, (NN, NG), 1)
    G = (n2g[...] == gid).astype(_f32)

    q_star = jnp.zeros((NG, 2 * H), _f32)
    h0 = jnp.zeros((NG, H), _f32)
    c0 = jnp.zeros((NG, H), _f32)
    h1 = jnp.zeros((NG, H), _f32)
    c1 = jnp.zeros((NG, H), _f32)
    for _ in range(NI):
        z = (jnp.dot(q_star, wi0[...], preferred_element_type=_f32)
             + jnp.dot(h0, wh0[...], preferred_element_type=_f32) + b0[...])
        zi, zf, zg, zo = z[:, :H], z[:, H:2*H], z[:, 2*H:3*H], z[:, 3*H:]
        c0 = _sigmoid(zf) * c0 + _sigmoid(zi) * jnp.tanh(zg)
        h0 = _sigmoid(zo) * jnp.tanh(c0)
        z = (jnp.dot(h0, wi1[...], preferred_element_type=_f32)
             + jnp.dot(h1, wh1[...], preferred_element_type=_f32) + b1[...])
        zi, zf, zg, zo = z[:, :H], z[:, H:2*H], z[:, 2*H:3*H], z[:, 3*H:]
        c1 = _sigmoid(zf) * c1 + _sigmoid(zi) * jnp.tanh(zg)
        h1 = _sigmoid(zo) * jnp.tanh(c1)
        q = h1
        qg = jnp.dot(G, q, preferred_element_type=_f32)
        logits = jnp.sum(h * qg, axis=1, keepdims=True)
        lmasked = jnp.where(G > 0.0, logits, -1e30)
        lmax = jnp.max(lmasked, axis=0, keepdims=True)
        lmax = jnp.where(lmax > -1e29, lmax, 0.0)
        lmax_pn = lax.dot_general(G, lmax, (((1,), (1,)), ((), ())),
                                  preferred_element_type=_f32)
        ex = jnp.exp(logits - lmax_pn)
        denom = lax.dot_general(G, ex, (((0,), (0,)), ((), ())),
                                preferred_element_type=_f32)
        denom_pn = jnp.dot(G, denom, preferred_element_type=_f32)
        alpha = ex / denom_pn
        r = lax.dot_general(G, alpha * h, (((0,), (0,)), ((), ())),
                            preferred_element_type=_f32)
        q_star = jnp.concatenate([q, r], axis=1)
    hg = jnp.maximum(
        jnp.dot(q_star, wm1[...], preferred_element_type=_f32) + bm1[...], 0.0)
    out[...] = jnp.dot(hg, wm2[...], preferred_element_type=_f32) + bm2[...]


def _final(h, acc, Wu, bu, n2g, Wi0, Wh0, b0, Wi1, Wh1, b1, Wm1, bm1,
           Wm2, bm2):
    return pl.pallas_call(
        _final_body,
        out_shape=jax.ShapeDtypeStruct((NG, 1), _f32),
    )(h, acc, Wu, bu, n2g, Wi0, Wh0, b0, Wi1, Wh1, b1, Wm1, bm1, Wm2, bm2)


def kernel(x_node, x_edge, edge_index, node2graph, W_node, b_node, W_edge,
           b_edge, W_msg, b_msg, W_upd, b_upd, Wi0, Wh0, b0, Wi1, Wh1, b1,
           Wm1, bm1, Wm2, bm2):
    src_r = edge_index[0].reshape(NW, NPASS, CPP, CH)
    dst_r = edge_index[1].reshape(NW, NPASS, CPP, CH)
    n2g = node2graph.reshape(NN, 1)
    zeros_nh = jnp.zeros((NN, H), _f32)

    A = [W_msg[l, :H, :] for l in range(NL)]
    Wm_e = W_msg[:, H:, :]
    bm = b_msg.reshape(NL, 1, H)

    ec = _edge_pre(x_edge, W_edge, b_edge.reshape(1, HE), Wm_e, bm)
    h, p = _node0(x_node, W_node, b_node.reshape(1, H), A[0])
    acc = None
    for l in range(NL):
        acc = _sc_edge(p, ec[l], src_r, dst_r, zeros_nh)
        if l < NL - 1:
            h, p = _upd(h, acc, W_upd[l], b_upd[l].reshape(1, H), A[l + 1])
    return _final(h, acc, W_upd[2], b_upd[2].reshape(1, H), n2g,
                  Wi0, Wh0, b0.reshape(1, 4 * H), Wi1, Wh1,
                  b1.reshape(1, 4 * H), Wm1, bm1.reshape(1, H),
                  Wm2, bm2.reshape(1, 1))


# speedup vs baseline: 4.7313x; 1.0684x over previous
import functools

import jax
import jax.numpy as jnp
import numpy as np
from jax import lax
from jax.experimental import pallas as pl
from jax.experimental.pallas import tpu as pltpu
from jax.experimental.pallas import tpu_sc as plsc

NN = 10000
NE = 320000
DN = 128
DE = 16
H = 128
HE = 16
NL = 3
NG = 64
NI = 6

NC = 2
NS = 16
NW = NC * NS
EPW = NE // NW
CH = 80
NCH = EPW // CH
NPASS = 5
CPP = NCH // NPASS
RPS = 640
RPS_LAST = NN - RPS * (NS - 1)

_f32 = jnp.float32

_PERM_LO = np.array([32 * (w // 16) + (w % 16) for w in range(H // 2)],
                    np.int32)
_PERM_HI = _PERM_LO + 16


def _sc_edge_body(p_hbm, ec_hbm, src_hbm, dst_hbm, zero_hbm, out_hbm,
                  src_v, dst_v, ec_v, g_v, acc_sh, ec_s0, ec_s1, g_s0, g_s1):
    c = lax.axis_index("c")
    s = lax.axis_index("s")
    wid = c * NS + s
    ebase = wid * EPW
    stripe = pl.multiple_of(s * RPS, 8)

    @pl.when(s < NS - 1)
    def _():
        pltpu.sync_copy(zero_hbm.at[pl.ds(stripe, RPS)],
                        acc_sh.at[pl.ds(stripe, RPS)])

    @pl.when(s == NS - 1)
    def _():
        pltpu.sync_copy(zero_hbm.at[pl.ds(RPS * (NS - 1), RPS_LAST)],
                        acc_sh.at[pl.ds(RPS * (NS - 1), RPS_LAST)])

    plsc.subcore_barrier()

    ec_sems = (ec_s0, ec_s1)
    g_sems = (g_s0, g_s1)

    def ppass(p, carry):
        pltpu.sync_copy(src_hbm.at[wid, p], src_v)
        pltpu.sync_copy(dst_hbm.at[wid, p], dst_v)
        pbase = ebase + p * (CPP * CH)

        def issue(j, b):
            off = pl.multiple_of(pbase + j * CH, 8)
            pltpu.async_copy(ec_hbm.at[pl.ds(off, CH)], ec_v.at[b],
                             ec_sems[b])
            pltpu.async_copy(p_hbm.at[src_v.at[j]], g_v.at[b], g_sems[b])

        def process(j, b):
            pltpu.make_async_copy(ec_hbm.at[pl.ds(ebase, CH)], ec_v.at[b],
                                  ec_sems[b]).wait()
            pltpu.make_async_copy(p_hbm.at[src_v.at[j]], g_v.at[b],
                                  g_sems[b]).wait()

            def row(r, cr):
                for q in range(H // 32):
                    vi = plsc.bitcast(ec_v[b, r, pl.ds(16 * q, 16)],
                                      jnp.int32)
                    a0 = plsc.bitcast(vi << 16, _f32)
                    a1 = plsc.bitcast(vi & jnp.int32(-65536), _f32)
                    s0 = pl.ds(32 * q, 16)
                    s1 = pl.ds(32 * q + 16, 16)
                    g_v[b, r, s0] = jnp.maximum(g_v[b, r, s0] + a0, 0.0)
                    g_v[b, r, s1] = jnp.maximum(g_v[b, r, s1] + a1, 0.0)
                return cr

            lax.fori_loop(0, CH, row, 0)
            pltpu.sync_copy(g_v.at[b], acc_sh.at[dst_v.at[j]], add=True)

            @pl.when(j + 2 < CPP)
            def _():
                issue(j + 2, b)

        issue(0, 0)
        issue(1, 1)

        def step(t, cr):
            process(2 * t, 0)
            process(2 * t + 1, 1)
            return cr

        lax.fori_loop(0, CPP // 2, step, 0)
        process(CPP - 1, (CPP - 1) % 2)
        return carry

    lax.fori_loop(0, NPASS, ppass, 0)
    plsc.subcore_barrier()

    @pl.when(s < NS - 1)
    def _():
        pltpu.sync_copy(acc_sh.at[pl.ds(stripe, RPS)],
                        out_hbm.at[c, pl.ds(stripe, RPS)])

    @pl.when(s == NS - 1)
    def _():
        pltpu.sync_copy(acc_sh.at[pl.ds(RPS * (NS - 1), RPS_LAST)],
                        out_hbm.at[c, pl.ds(RPS * (NS - 1), RPS_LAST)])


@functools.cache
def _sc_edge_kernel():
    return pl.kernel(
        _sc_edge_body,
        out_type=jax.ShapeDtypeStruct((NC, NN, H), _f32),
        mesh=plsc.VectorSubcoreMesh(core_axis_name="c", subcore_axis_name="s",
                                    num_cores=NC, num_subcores=NS),
        scratch_types=[
            pltpu.VMEM((CPP, CH), jnp.int32),
            pltpu.VMEM((CPP, CH), jnp.int32),
            pltpu.VMEM((2, CH, H // 2), _f32),
            pltpu.VMEM((2, CH, H), _f32),
            pltpu.VMEM_SHARED((NN, H), _f32),
            pltpu.SemaphoreType.DMA,
            pltpu.SemaphoreType.DMA,
            pltpu.SemaphoreType.DMA,
            pltpu.SemaphoreType.DMA,
        ],
        compiler_params=pltpu.CompilerParams(needs_layout_passes=False),
    )


def _sc_edge(p, ec_l, src_r, dst_r, zeros_nh):
    return _sc_edge_kernel()(p, ec_l, src_r, dst_r, zeros_nh)


_BE = 4000


def _rne_bf16_bits(x):
    i = lax.bitcast_convert_type(x, jnp.int32)
    return i + 0x7FFF + ((i >> 16) & 1)


def _edge_pre_body(xe, we, be, wma, bma, wmb, bmb, out):
    e = jnp.maximum(
        jnp.dot(xe[...], we[...], preferred_element_type=_f32) + be[...], 0.0)
    for l in range(NL):
        eca = jnp.dot(e, wma[l], preferred_element_type=_f32) + bma[l]
        ecb = jnp.dot(e, wmb[l], preferred_element_type=_f32) + bmb[l]
        word = (lax.shift_right_logical(_rne_bf16_bits(eca), 16)
                | (_rne_bf16_bits(ecb) & jnp.int32(-65536)))
        out[l] = lax.bitcast_convert_type(word, jnp.float32)


def _edge_pre(x_edge, W_edge, b_edge, WmA, bmA, WmB, bmB):
    return pl.pallas_call(
        _edge_pre_body,
        grid=(NE // _BE,),
        in_specs=[
            pl.BlockSpec((_BE, DE), lambda i: (i, 0)),
            pl.BlockSpec((DE, HE), lambda i: (0, 0)),
            pl.BlockSpec((1, HE), lambda i: (0, 0)),
            pl.BlockSpec((NL, HE, H // 2), lambda i: (0, 0, 0)),
            pl.BlockSpec((NL, 1, H // 2), lambda i: (0, 0, 0)),
            pl.BlockSpec((NL, HE, H // 2), lambda i: (0, 0, 0)),
            pl.BlockSpec((NL, 1, H // 2), lambda i: (0, 0, 0)),
        ],
        out_specs=pl.BlockSpec((NL, _BE, H // 2), lambda i: (0, i, 0)),
        out_shape=jax.ShapeDtypeStruct((NL, NE, H // 2), _f32),
    )(x_edge, W_edge, b_edge, WmA, bmA, WmB, bmB)


_BN = 2000


def _node0_body(xn, wn, bn, a, h_out, p_out):
    h = jnp.maximum(
        jnp.dot(xn[...], wn[...], preferred_element_type=_f32) + bn[...], 0.0)
    h_out[...] = h
    p_out[...] = jnp.dot(h, a[...], preferred_element_type=_f32)


def _node0(x_node, W_node, b_node, A0):
    return pl.pallas_call(
        _node0_body,
        grid=(NN // _BN,),
        in_specs=[
            pl.BlockSpec((_BN, DN), lambda i: (i, 0)),
            pl.BlockSpec((DN, H), lambda i: (0, 0)),
            pl.BlockSpec((1, H), lambda i: (0, 0)),
            pl.BlockSpec((H, H), lambda i: (0, 0)),
        ],
        out_specs=[
            pl.BlockSpec((_BN, H), lambda i: (i, 0)),
            pl.BlockSpec((_BN, H), lambda i: (i, 0)),
        ],
        out_shape=[
            jax.ShapeDtypeStruct((NN, H), _f32),
            jax.ShapeDtypeStruct((NN, H), _f32),
        ],
    )(x_node, W_node, b_node, A0)


def _upd_body(h, acc, wu, bu, a, hn_out, p_out):
    agg = acc[0] + acc[1]
    hn = jnp.maximum(
        h[...] + jnp.dot(agg, wu[...], preferred_element_type=_f32) + bu[...],
        0.0)
    hn_out[...] = hn
    p_out[...] = jnp.dot(hn, a[...], preferred_element_type=_f32)


def _upd(h, acc, Wu, bu, Anext):
    return pl.pallas_call(
        _upd_body,
        grid=(NN // _BN,),
        in_specs=[
            pl.BlockSpec((_BN, H), lambda i: (i, 0)),
            pl.BlockSpec((NC, _BN, H), lambda i: (0, i, 0)),
            pl.BlockSpec((H, H), lambda i: (0, 0)),
            pl.BlockSpec((1, H), lambda i: (0, 0)),
            pl.BlockSpec((H, H), lambda i: (0, 0)),
        ],
        out_specs=[
            pl.BlockSpec((_BN, H), lambda i: (i, 0)),
            pl.BlockSpec((_BN, H), lambda i: (i, 0)),
        ],
        out_shape=[
            jax.ShapeDtypeStruct((NN, H), _f32),
            jax.ShapeDtypeStruct((NN, H), _f32),
        ],
    )(h, acc, Wu, bu, Anext)


def _sigmoid(x):
    return 1.0 / (1.0 + jnp.exp(-x))


def _final_body(h_in, acc, wu, bu, n2g, wi0, wh0, b0, wi1, wh1, b1,
                wm1, bm1, wm2, bm2, out):
    agg = acc[0] + acc[1]
    h = jnp.maximum(
        h_in[...] + jnp.dot(agg, wu[...], preferred_element_type=_f32)
        + bu[...], 0.0)
    gid = lax.broadcasted_iota(jnp.int32, (NN, NG), 1)
    G = (n2g[...] == gid).astype(_f32)

    q_star = jnp.zeros((NG, 2 * H), _f32)
    h0 = jnp.zeros((NG, H), _f32)
    c0 = jnp.zeros((NG, H), _f32)
    h1 = jnp.zeros((NG, H), _f32)
    c1 = jnp.zeros((NG, H), _f32)
    for _ in range(NI):
        z = (jnp.dot(q_star, wi0[...], preferred_element_type=_f32)
             + jnp.dot(h0, wh0[...], preferred_element_type=_f32) + b0[...])
        zi, zf, zg, zo = z[:, :H], z[:, H:2*H], z[:, 2*H:3*H], z[:, 3*H:]
        c0 = _sigmoid(zf) * c0 + _sigmoid(zi) * jnp.tanh(zg)
        h0 = _sigmoid(zo) * jnp.tanh(c0)
        z = (jnp.dot(h0, wi1[...], preferred_element_type=_f32)
             + jnp.dot(h1, wh1[...], preferred_element_type=_f32) + b1[...])
        zi, zf, zg, zo = z[:, :H], z[:, H:2*H], z[:, 2*H:3*H], z[:, 3*H:]
        c1 = _sigmoid(zf) * c1 + _sigmoid(zi) * jnp.tanh(zg)
        h1 = _sigmoid(zo) * jnp.tanh(c1)
        q = h1
        qg = jnp.dot(G, q, preferred_element_type=_f32)
        logits = jnp.sum(h * qg, axis=1, keepdims=True)
        lmasked = jnp.where(G > 0.0, logits, -1e30)
        lmax = jnp.max(lmasked, axis=0, keepdims=True)
        lmax = jnp.where(lmax > -1e29, lmax, 0.0)
        lmax_pn = lax.dot_general(G, lmax, (((1,), (1,)), ((), ())),
                                  preferred_element_type=_f32)
        ex = jnp.exp(logits - lmax_pn)
        denom = lax.dot_general(G, ex, (((0,), (0,)), ((), ())),
                                preferred_element_type=_f32)
        denom_pn = jnp.dot(G, denom, preferred_element_type=_f32)
        alpha = ex / denom_pn
        r = lax.dot_general(G, alpha * h, (((0,), (0,)), ((), ())),
                            preferred_element_type=_f32)
        q_star = jnp.concatenate([q, r], axis=1)
    hg = jnp.maximum(
        jnp.dot(q_star, wm1[...], preferred_element_type=_f32) + bm1[...], 0.0)
    out[...] = jnp.dot(hg, wm2[...], preferred_element_type=_f32) + bm2[...]


def _final(h, acc, Wu, bu, n2g, Wi0, Wh0, b0, Wi1, Wh1, b1, Wm1, bm1,
           Wm2, bm2):
    return pl.pallas_call(
        _final_body,
        out_shape=jax.ShapeDtypeStruct((NG, 1), _f32),
    )(h, acc, Wu, bu, n2g, Wi0, Wh0, b0, Wi1, Wh1, b1, Wm1, bm1, Wm2, bm2)


def kernel(x_node, x_edge, edge_index, node2graph, W_node, b_node, W_edge,
           b_edge, W_msg, b_msg, W_upd, b_upd, Wi0, Wh0, b0, Wi1, Wh1, b1,
           Wm1, bm1, Wm2, bm2):
    src_r = edge_index[0].reshape(NW, NPASS, CPP, CH)
    dst_r = edge_index[1].reshape(NW, NPASS, CPP, CH)
    n2g = node2graph.reshape(NN, 1)
    zeros_nh = jnp.zeros((NN, H), _f32)

    A = [W_msg[l, :H, :] for l in range(NL)]
    Wm_e = W_msg[:, H:, :]
    WmA = Wm_e[:, :, _PERM_LO]
    WmB = Wm_e[:, :, _PERM_HI]
    bmA = b_msg[:, _PERM_LO].reshape(NL, 1, H // 2)
    bmB = b_msg[:, _PERM_HI].reshape(NL, 1, H // 2)

    ec = _edge_pre(x_edge, W_edge, b_edge.reshape(1, HE), WmA, bmA, WmB, bmB)
    h, p = _node0(x_node, W_node, b_node.reshape(1, H), A[0])
    acc = None
    for l in range(NL):
        acc = _sc_edge(p, ec[l], src_r, dst_r, zeros_nh)
        if l < NL - 1:
            h, p = _upd(h, acc, W_upd[l], b_upd[l].reshape(1, H), A[l + 1])
    return _final(h, acc, W_upd[2], b_upd[2].reshape(1, H), n2g,
                  Wi0, Wh0, b0.reshape(1, 4 * H), Wi1, Wh1,
                  b1.reshape(1, 4 * H), Wm1, bm1.reshape(1, H),
                  Wm2, bm2.reshape(1, 1))


# speedup vs baseline: 6.3335x; 1.3386x over previous
import functools

import jax
import jax.numpy as jnp
import numpy as np
from jax import lax
from jax.experimental import pallas as pl
from jax.experimental.pallas import tpu as pltpu
from jax.experimental.pallas import tpu_sc as plsc

NN = 10000
NE = 320000
DN = 128
DE = 16
H = 128
HE = 16
NL = 3
NG = 64
NI = 6

NC = 2
NS = 16
NW = NC * NS
EPW = NE // NW
CH = 80
NCH = EPW // CH
NPASS = 5
CPP = NCH // NPASS
RPS = 640
RPS_LAST = NN - RPS * (NS - 1)

_f32 = jnp.float32

_PERM_LO = np.array([32 * (w // 16) + (w % 16) for w in range(H // 2)],
                    np.int32)
_PERM_HI = _PERM_LO + 16


def _sc_edge_body(lidx, p_hbm, ec_hbm, idx_hbm, zero_hbm, out_hbm,
                  src_v, dst_v, ec_v, g_v, acc_sh, ec_s0, ec_s1, g_s0, g_s1):
    c = lax.axis_index("c")
    s = lax.axis_index("s")
    wid = c * NS + s
    ebase = wid * EPW
    stripe = pl.multiple_of(s * RPS, 8)

    @pl.when(s < NS - 1)
    def _():
        pltpu.sync_copy(zero_hbm.at[pl.ds(stripe, RPS)],
                        acc_sh.at[pl.ds(stripe, RPS)])

    @pl.when(s == NS - 1)
    def _():
        pltpu.sync_copy(zero_hbm.at[pl.ds(RPS * (NS - 1), RPS_LAST)],
                        acc_sh.at[pl.ds(RPS * (NS - 1), RPS_LAST)])

    plsc.subcore_barrier()

    ec_sems = (ec_s0, ec_s1)
    g_sems = (g_s0, g_s1)

    def ppass(p, carry):
        pltpu.sync_copy(idx_hbm.at[0, wid, p], src_v)
        pltpu.sync_copy(idx_hbm.at[1, wid, p], dst_v)
        pbase = ebase + p * (CPP * CH)

        def issue(j, b):
            off = pl.multiple_of(pbase + j * CH, 8)
            pltpu.async_copy(ec_hbm.at[lidx, pl.ds(off, CH)], ec_v.at[b],
                             ec_sems[b])
            pltpu.async_copy(p_hbm.at[src_v.at[j]], g_v.at[b], g_sems[b])

        def process(j, b):
            pltpu.make_async_copy(ec_hbm.at[lidx, pl.ds(ebase, CH)],
                                  ec_v.at[b], ec_sems[b]).wait()
            pltpu.make_async_copy(p_hbm.at[src_v.at[j]], g_v.at[b],
                                  g_sems[b]).wait()

            def row(r, cr):
                for q in range(H // 32):
                    vi = plsc.bitcast(ec_v[b, r, pl.ds(16 * q, 16)],
                                      jnp.int32)
                    a0 = plsc.bitcast(vi << 16, _f32)
                    a1 = plsc.bitcast(vi & jnp.int32(-65536), _f32)
                    s0 = pl.ds(32 * q, 16)
                    s1 = pl.ds(32 * q + 16, 16)
                    g_v[b, r, s0] = jnp.maximum(g_v[b, r, s0] + a0, 0.0)
                    g_v[b, r, s1] = jnp.maximum(g_v[b, r, s1] + a1, 0.0)
                return cr

            lax.fori_loop(0, CH, row, 0)
            pltpu.sync_copy(g_v.at[b], acc_sh.at[dst_v.at[j]], add=True)

            @pl.when(j + 2 < CPP)
            def _():
                issue(j + 2, b)

        issue(0, 0)
        issue(1, 1)

        def step(t, cr):
            process(2 * t, 0)
            process(2 * t + 1, 1)
            return cr

        lax.fori_loop(0, CPP // 2, step, 0)
        process(CPP - 1, (CPP - 1) % 2)
        return carry

    lax.fori_loop(0, NPASS, ppass, 0)
    plsc.subcore_barrier()

    @pl.when(s < NS - 1)
    def _():
        pltpu.sync_copy(acc_sh.at[pl.ds(stripe, RPS)],
                        out_hbm.at[c, pl.ds(stripe, RPS)])

    @pl.when(s == NS - 1)
    def _():
        pltpu.sync_copy(acc_sh.at[pl.ds(RPS * (NS - 1), RPS_LAST)],
                        out_hbm.at[c, pl.ds(RPS * (NS - 1), RPS_LAST)])


@functools.cache
def _sc_edge_kernel(lidx):
    return pl.kernel(
        functools.partial(_sc_edge_body, lidx),
        out_type=jax.ShapeDtypeStruct((NC, NN, H), _f32),
        mesh=plsc.VectorSubcoreMesh(core_axis_name="c", subcore_axis_name="s",
                                    num_cores=NC, num_subcores=NS),
        scratch_types=[
            pltpu.VMEM((CPP, CH), jnp.int32),
            pltpu.VMEM((CPP, CH), jnp.int32),
            pltpu.VMEM((2, CH, H // 2), _f32),
            pltpu.VMEM((2, CH, H), _f32),
            pltpu.VMEM_SHARED((NN, H), _f32),
            pltpu.SemaphoreType.DMA,
            pltpu.SemaphoreType.DMA,
            pltpu.SemaphoreType.DMA,
            pltpu.SemaphoreType.DMA,
        ],
        compiler_params=pltpu.CompilerParams(needs_layout_passes=False),
    )


def _sc_edge(lidx, p, ec_full, idx_r, zeros_nh):
    return _sc_edge_kernel(lidx)(p, ec_full, idx_r, zeros_nh)


_BE = 4000


def _rne_bf16_bits(x):
    i = lax.bitcast_convert_type(x, jnp.int32)
    return i + 0x7FFF + ((i >> 16) & 1)


def _edge_pre_body(xe, we, be, wma, bma, wmb, bmb, out):
    e = jnp.maximum(
        jnp.dot(xe[...], we[...], preferred_element_type=_f32) + be[...], 0.0)
    eca = jnp.dot(e, wma[...], preferred_element_type=_f32) + bma[...]
    ecb = jnp.dot(e, wmb[...], preferred_element_type=_f32) + bmb[...]
    word = (lax.shift_right_logical(_rne_bf16_bits(eca), 16)
            | (_rne_bf16_bits(ecb) & jnp.int32(-65536)))
    w32 = lax.bitcast_convert_type(word, jnp.float32)
    for l in range(NL):
        out[l] = w32[:, l * (H // 2):(l + 1) * (H // 2)]


def _edge_pre(x_edge, W_edge, b_edge, WmA, bmA, WmB, bmB):
    return pl.pallas_call(
        _edge_pre_body,
        grid=(NE // _BE,),
        in_specs=[
            pl.BlockSpec((_BE, DE), lambda i: (i, 0)),
            pl.BlockSpec((DE, HE), lambda i: (0, 0)),
            pl.BlockSpec((1, HE), lambda i: (0, 0)),
            pl.BlockSpec((HE, NL * H // 2), lambda i: (0, 0)),
            pl.BlockSpec((1, NL * H // 2), lambda i: (0, 0)),
            pl.BlockSpec((HE, NL * H // 2), lambda i: (0, 0)),
            pl.BlockSpec((1, NL * H // 2), lambda i: (0, 0)),
        ],
        out_specs=pl.BlockSpec((NL, _BE, H // 2), lambda i: (0, i, 0)),
        out_shape=jax.ShapeDtypeStruct((NL, NE, H // 2), _f32),
    )(x_edge, W_edge, b_edge, WmA, bmA, WmB, bmB)


_BN = 2000


def _node0_body(xn, wn, bn, a, h_out, p_out):
    h = jnp.maximum(
        jnp.dot(xn[...], wn[...], preferred_element_type=_f32) + bn[...], 0.0)
    h_out[...] = h
    p_out[...] = jnp.dot(h, a[...], preferred_element_type=_f32)


def _node0(x_node, W_node, b_node, A0):
    return pl.pallas_call(
        _node0_body,
        grid=(NN // _BN,),
        in_specs=[
            pl.BlockSpec((_BN, DN), lambda i: (i, 0)),
            pl.BlockSpec((DN, H), lambda i: (0, 0)),
            pl.BlockSpec((1, H), lambda i: (0, 0)),
            pl.BlockSpec((H, H), lambda i: (0, 0)),
        ],
        out_specs=[
            pl.BlockSpec((_BN, H), lambda i: (i, 0)),
            pl.BlockSpec((_BN, H), lambda i: (i, 0)),
        ],
        out_shape=[
            jax.ShapeDtypeStruct((NN, H), _f32),
            jax.ShapeDtypeStruct((NN, H), _f32),
        ],
    )(x_node, W_node, b_node, A0)


def _upd_body(h, acc, wu, bu, a, hn_out, p_out):
    agg = acc[0] + acc[1]
    hn = jnp.maximum(
        h[...] + jnp.dot(agg, wu[...], preferred_element_type=_f32) + bu[...],
        0.0)
    hn_out[...] = hn
    p_out[...] = jnp.dot(hn, a[...], preferred_element_type=_f32)


def _upd(h, acc, Wu, bu, Anext):
    return pl.pallas_call(
        _upd_body,
        grid=(NN // _BN,),
        in_specs=[
            pl.BlockSpec((_BN, H), lambda i: (i, 0)),
            pl.BlockSpec((NC, _BN, H), lambda i: (0, i, 0)),
            pl.BlockSpec((H, H), lambda i: (0, 0)),
            pl.BlockSpec((1, H), lambda i: (0, 0)),
            pl.BlockSpec((H, H), lambda i: (0, 0)),
        ],
        out_specs=[
            pl.BlockSpec((_BN, H), lambda i: (i, 0)),
            pl.BlockSpec((_BN, H), lambda i: (i, 0)),
        ],
        out_shape=[
            jax.ShapeDtypeStruct((NN, H), _f32),
            jax.ShapeDtypeStruct((NN, H), _f32),
        ],
    )(h, acc, Wu, bu, Anext)


def _sigmoid(x):
    return 1.0 / (1.0 + jnp.exp(-x))


def _final_body(h_in, acc, wu, bu, n2g, wi0, wh0, b0, wi1, wh1, b1,
                wm1, bm1, wm2, bm2, out):
    agg = acc[0] + acc[1]
    h = jnp.maximum(
        h_in[...] + jnp.dot(agg, wu[...], preferred_element_type=_f32)
        + bu[...], 0.0)
    gid = lax.broadcasted_iota(jnp.int32, (NN, NG), 1)
    G = (n2g[...] == gid).astype(_f32)

    q_star = jnp.zeros((NG, 2 * H), _f32)
    h0 = jnp.zeros((NG, H), _f32)
    c0 = jnp.zeros((NG, H), _f32)
    h1 = jnp.zeros((NG, H), _f32)
    c1 = jnp.zeros((NG, H), _f32)
    for _ in range(NI):
        z = (jnp.dot(q_star, wi0[...], preferred_element_type=_f32)
             + jnp.dot(h0, wh0[...], preferred_element_type=_f32) + b0[...])
        zi, zf, zg, zo = z[:, :H], z[:, H:2*H], z[:, 2*H:3*H], z[:, 3*H:]
        c0 = _sigmoid(zf) * c0 + _sigmoid(zi) * jnp.tanh(zg)
        h0 = _sigmoid(zo) * jnp.tanh(c0)
        z = (jnp.dot(h0, wi1[...], preferred_element_type=_f32)
             + jnp.dot(h1, wh1[...], preferred_element_type=_f32) + b1[...])
        zi, zf, zg, zo = z[:, :H], z[:, H:2*H], z[:, 2*H:3*H], z[:, 3*H:]
        c1 = _sigmoid(zf) * c1 + _sigmoid(zi) * jnp.tanh(zg)
        h1 = _sigmoid(zo) * jnp.tanh(c1)
        q = h1
        qg = jnp.dot(G, q, preferred_element_type=_f32)
        logits = jnp.sum(h * qg, axis=1, keepdims=True)
        lmasked = jnp.where(G > 0.0, logits, -1e30)
        lmax = jnp.max(lmasked, axis=0, keepdims=True)
        lmax = jnp.where(lmax > -1e29, lmax, 0.0)
        lmax_pn = lax.dot_general(G, lmax, (((1,), (1,)), ((), ())),
                                  preferred_element_type=_f32)
        ex = jnp.exp(logits - lmax_pn)
        denom = lax.dot_general(G, ex, (((0,), (0,)), ((), ())),
                                preferred_element_type=_f32)
        denom_pn = jnp.dot(G, denom, preferred_element_type=_f32)
        alpha = ex / denom_pn
        r = lax.dot_general(G, alpha * h, (((0,), (0,)), ((), ())),
                            preferred_element_type=_f32)
        q_star = jnp.concatenate([q, r], axis=1)
    hg = jnp.maximum(
        jnp.dot(q_star, wm1[...], preferred_element_type=_f32) + bm1[...], 0.0)
    out[...] = jnp.dot(hg, wm2[...], preferred_element_type=_f32) + bm2[...]


def _final(h, acc, Wu, bu, n2g, Wi0, Wh0, b0, Wi1, Wh1, b1, Wm1, bm1,
           Wm2, bm2):
    return pl.pallas_call(
        _final_body,
        out_shape=jax.ShapeDtypeStruct((NG, 1), _f32),
    )(h, acc, Wu, bu, n2g, Wi0, Wh0, b0, Wi1, Wh1, b1, Wm1, bm1, Wm2, bm2)


def kernel(x_node, x_edge, edge_index, node2graph, W_node, b_node, W_edge,
           b_edge, W_msg, b_msg, W_upd, b_upd, Wi0, Wh0, b0, Wi1, Wh1, b1,
           Wm1, bm1, Wm2, bm2):
    idx_r = edge_index.reshape(2, NW, NPASS, CPP, CH)
    n2g = node2graph.reshape(NN, 1)
    zeros_nh = jnp.zeros((NN, H), _f32)

    A = [W_msg[l, :H, :] for l in range(NL)]
    Wm_e = W_msg[:, H:, :]
    WmA = jnp.concatenate([Wm_e[l][:, _PERM_LO] for l in range(NL)], axis=1)
    WmB = jnp.concatenate([Wm_e[l][:, _PERM_HI] for l in range(NL)], axis=1)
    bmA = jnp.concatenate([b_msg[l][_PERM_LO] for l in range(NL)]).reshape(
        1, NL * H // 2)
    bmB = jnp.concatenate([b_msg[l][_PERM_HI] for l in range(NL)]).reshape(
        1, NL * H // 2)

    ec = _edge_pre(x_edge, W_edge, b_edge.reshape(1, HE), WmA, bmA, WmB, bmB)
    h, p = _node0(x_node, W_node, b_node.reshape(1, H), A[0])
    acc = None
    for l in range(NL):
        acc = _sc_edge(l, p, ec, idx_r, zeros_nh)
        if l < NL - 1:
            h, p = _upd(h, acc, W_upd[l], b_upd[l].reshape(1, H), A[l + 1])
    return _final(h, acc, W_upd[2], b_upd[2].reshape(1, H), n2g,
                  Wi0, Wh0, b0.reshape(1, 4 * H), Wi1, Wh1,
                  b1.reshape(1, 4 * H), Wm1, bm1.reshape(1, H),
                  Wm2, bm2.reshape(1, 1))
